# Initial kernel scaffold; baseline (speedup 1.0000x reference)
import jax
import jax.numpy as jnp
from jax.experimental import pallas as pl


def kernel(x, edge_index, W1, b1, W2, b2):
    raise NotImplementedError("write your pallas kernel here")


# speedup vs baseline: 4.0214x; 4.0214x over previous
import functools

import jax
import jax.numpy as jnp
from jax import lax
from jax.experimental import pallas as pl
from jax.experimental.pallas import tpu as pltpu
from jax.experimental.pallas import tpu_sc as plsc

NC = 2
NS = 16
L = 16
K = 128
RB = 2048


@functools.cache
def _degree_kernel(npad: int, epad: int):
    ed = epad // (NC * NS)

    @functools.partial(
        pl.kernel,
        out_type=jax.ShapeDtypeStruct((2, NC * NS, npad), jnp.float32),
        mesh=plsc.VectorSubcoreMesh(core_axis_name="c", subcore_axis_name="s"),
        compiler_params=pltpu.CompilerParams(needs_layout_passes=False),
        scratch_types=[
            pltpu.VMEM((ed,), jnp.int32),
            pltpu.VMEM((ed,), jnp.int32),
            pltpu.VMEM((npad,), jnp.float32),
            pltpu.VMEM((npad,), jnp.float32),
        ],
    )
    def deg(src_hbm, dst_hbm, out_hbm, src_v, dst_v, hsrc, hdst):
        c = lax.axis_index("c")
        s = lax.axis_index("s")
        w = s * NC + c
        pltpu.sync_copy(src_hbm.at[pl.ds(w * ed, ed)], src_v)
        pltpu.sync_copy(dst_hbm.at[pl.ds(w * ed, ed)], dst_v)

        zero = jnp.zeros((L,), jnp.float32)

        def zbody(i, _):
            hsrc[pl.ds(i * L, L)] = zero
            hdst[pl.ds(i * L, L)] = zero
            return 0

        lax.fori_loop(0, npad // L, zbody, 0)

        ones = jnp.full((L,), 1.0, jnp.float32)

        def ebody(i, _):
            plsc.addupdate_scatter(hsrc, [src_v[pl.ds(i * L, L)]], ones)
            plsc.addupdate_scatter(hdst, [dst_v[pl.ds(i * L, L)]], ones)
            return 0

        lax.fori_loop(0, ed // L, ebody, 0)

        pltpu.sync_copy(hsrc, out_hbm.at[0, w])
        pltpu.sync_copy(hdst, out_hbm.at[1, w])

    return deg


@functools.cache
def _edge_pass_kernel(npad: int, f: int, ech: int, feature_split: bool):
    rpw = npad // NS
    sch = 16
    assert ech % sch == 0

    @functools.partial(
        pl.kernel,
        out_type=jax.ShapeDtypeStruct((NC, npad, f), jnp.float32),
        mesh=plsc.VectorSubcoreMesh(core_axis_name="c", subcore_axis_name="s"),
        compiler_params=pltpu.CompilerParams(needs_layout_passes=False),
        scratch_types=[
            pltpu.VMEM((sch * K,), jnp.int32),
            pltpu.VMEM((sch, K), jnp.int32),
            pltpu.VMEM((K, f), jnp.float32),
            pltpu.VMEM((K, f), jnp.float32),
            pltpu.VMEM_SHARED((npad, f), jnp.float32),
            pltpu.SemaphoreType.DMA,
            pltpu.SemaphoreType.DMA,
        ],
    )
    def ep(tbl_hbm, srcb_hbm, dst2_hbm, out_hbm,
           src_v, dst_v, buf0, buf1, acc, gsem, ssem):
        c = lax.axis_index("c")
        s = lax.axis_index("s")
        if feature_split:
            row, cbase = c, s * ech
        else:
            row, cbase = 0, (s * NC + c) * ech

        zero = jnp.zeros((L,), jnp.float32)

        def zbody(i, _):
            for k in range(f // L):
                buf0[i, pl.ds(k * L, L)] = zero
            return 0

        lax.fori_loop(0, K, zbody, 0)
        for t in range(rpw // K):
            pltpu.sync_copy(buf0, acc.at[pl.ds(s * rpw + t * K, K)])
        plsc.subcore_barrier()

        def g_start(j, buf):
            pltpu.async_copy(tbl_hbm.at[src_v.at[pl.ds(j * K, K)]], buf, gsem)

        def g_wait(j, buf):
            pltpu.make_async_copy(
                tbl_hbm.at[src_v.at[pl.ds(j * K, K)]], buf, gsem).wait()

        def s_start(j, buf):
            pltpu.async_copy(buf, acc.at[dst_v.at[j]], ssem, add=True)

        def s_wait(j, buf):
            pltpu.make_async_copy(buf, acc.at[dst_v.at[j]], ssem).wait()

        def superchunk(m, _):
            base = cbase + m * sch
            pltpu.sync_copy(srcb_hbm.at[row, pl.ds(base * K, sch * K)], src_v)
            pltpu.sync_copy(dst2_hbm.at[pl.ds(base, sch)], dst_v)
            g_start(0, buf0)
            g_wait(0, buf0)
            s_start(0, buf0)
            g_start(1, buf1)
            for p in range((sch - 2) // 2):
                j1 = 2 * p + 1
                g_wait(j1, buf1)
                s_start(j1, buf1)
                s_wait(j1 - 1, buf0)
                g_start(j1 + 1, buf0)
                j2 = 2 * p + 2
                g_wait(j2, buf0)
                s_start(j2, buf0)
                s_wait(j2 - 1, buf1)
                g_start(j2 + 1, buf1)
            jl = sch - 1
            g_wait(jl, buf1)
            s_start(jl, buf1)
            s_wait(jl - 1, buf0)
            s_wait(jl, buf1)
            return 0

        lax.fori_loop(0, ech // sch, superchunk, 0)

        plsc.subcore_barrier()
        pltpu.sync_copy(acc.at[pl.ds(s * rpw, rpw)],
                        out_hbm.at[c, pl.ds(s * rpw, rpw)])

    return ep


def _run_degree(src_p, dst_p, npad, epad):
    return _degree_kernel(npad, epad)(src_p, dst_p)


def _run_edge_pass(tbl_flat, srcb, dst2, npad, f, ech, feature_split):
    return _edge_pass_kernel(npad, f, ech, feature_split)(tbl_flat, srcb, dst2)


def _prep_body(degp_ref, x_ref, xs_ref, ns_ref, nd_ref):
    d = jnp.sum(degp_ref[...], axis=1)
    ns = jnp.where(d[0] > 0, lax.rsqrt(d[0]), 0.0)
    nd = jnp.where(d[1] > 0, lax.rsqrt(d[1]), 0.0)
    ns_ref[...] = ns[:, None]
    nd_ref[...] = nd[:, None]
    xs_ref[...] = x_ref[...] * ns[:, None]


def _run_prep(degp, xp, npad, fin):
    nw = degp.shape[1]
    return pl.pallas_call(
        _prep_body,
        grid=(npad // RB,),
        in_specs=[
            pl.BlockSpec((2, nw, RB), lambda i: (0, 0, i)),
            pl.BlockSpec((RB, fin), lambda i: (i, 0)),
        ],
        out_specs=[
            pl.BlockSpec((RB, fin), lambda i: (i, 0)),
            pl.BlockSpec((RB, 1), lambda i: (i, 0)),
            pl.BlockSpec((RB, 1), lambda i: (i, 0)),
        ],
        out_shape=[
            jax.ShapeDtypeStruct((npad, fin), jnp.float32),
            jax.ShapeDtypeStruct((npad, 1), jnp.float32),
            jax.ShapeDtypeStruct((npad, 1), jnp.float32),
        ],
    )(degp, xp)


def _l1_body(a_ref, w_ref, b_ref, nd_ref, ns_ref, hs_ref):
    a = a_ref[0] + a_ref[1]
    acc = jnp.dot(a, w_ref[...], preferred_element_type=jnp.float32)
    h = acc * nd_ref[...] + b_ref[...]
    h = jnp.maximum(h, 0.0) * ns_ref[...]
    hh = h.shape[1] // 2
    hs_ref[0] = h[:, :hh]
    hs_ref[1] = h[:, hh:]


def _run_l1(agg, w1, b1, nd, ns, npad, fin, hid):
    return pl.pallas_call(
        _l1_body,
        grid=(npad // RB,),
        in_specs=[
            pl.BlockSpec((2, RB, fin), lambda i: (0, i, 0)),
            pl.BlockSpec((fin, hid), lambda i: (0, 0)),
            pl.BlockSpec((1, hid), lambda i: (0, 0)),
            pl.BlockSpec((RB, 1), lambda i: (i, 0)),
            pl.BlockSpec((RB, 1), lambda i: (i, 0)),
        ],
        out_specs=pl.BlockSpec((2, RB, hid // 2), lambda i: (0, i, 0)),
        out_shape=jax.ShapeDtypeStruct((2, npad, hid // 2), jnp.float32),
    )(agg, w1, b1, nd, ns)


def _l2_body(a_ref, w_ref, b_ref, nd_ref, out_ref):
    w = w_ref[...]
    half = a_ref.shape[2]
    acc = (jnp.dot(a_ref[0], w[:half], preferred_element_type=jnp.float32)
           + jnp.dot(a_ref[1], w[half:], preferred_element_type=jnp.float32))
    out_ref[...] = acc * nd_ref[...] + b_ref[...]


def _run_l2(agg, w2, b2, nd, npad, hid):
    return pl.pallas_call(
        _l2_body,
        grid=(npad // RB,),
        in_specs=[
            pl.BlockSpec((2, RB, hid // 2), lambda i: (0, i, 0)),
            pl.BlockSpec((hid, hid), lambda i: (0, 0)),
            pl.BlockSpec((1, hid), lambda i: (0, 0)),
            pl.BlockSpec((RB, 1), lambda i: (i, 0)),
        ],
        out_specs=pl.BlockSpec((RB, hid), lambda i: (i, 0)),
        out_shape=jax.ShapeDtypeStruct((npad, hid), jnp.float32),
    )(agg, w2, b2, nd)


def kernel(x, edge_index, W1, b1, W2, b2):
    n, fin = x.shape
    hid = W1.shape[1]
    e = edge_index.shape[1]

    npad = -(-n // RB) * RB
    grp = 8 * K * NS * NC
    epad = -(-e // grp) * grp
    ech1 = epad // (K * NS * NC)
    ech2 = epad // (K * NS)

    src = edge_index[0].astype(jnp.int32)
    dst = edge_index[1].astype(jnp.int32)
    fill = jnp.full((epad - e,), npad - 1, jnp.int32)
    src_p = jnp.concatenate([src, fill])
    dst_p = jnp.concatenate([dst, fill])
    srcb = jnp.stack([src_p, src_p + npad])
    dst2 = dst_p.reshape(epad // K, K)
    xp = jnp.pad(x, ((0, npad - n), (0, 0)))

    degp = _run_degree(src_p, dst_p, npad, epad)
    xs, ns, nd = _run_prep(degp, xp, npad, fin)
    agg1 = _run_edge_pass(xs, srcb, dst2, npad, fin, ech1, False)
    hs = _run_l1(agg1, W1, b1.reshape(1, hid), nd, ns, npad, fin, hid)
    agg2 = _run_edge_pass(hs.reshape(NC * npad, hid // 2), srcb, dst2,
                          npad, hid // 2, ech2, True)
    out = _run_l2(agg2, W2, b2.reshape(1, hid), nd, npad, hid)
    return out[:n]


# speedup vs baseline: 9.8379x; 2.4464x over previous
import functools

import jax
import jax.numpy as jnp
from jax import lax
from jax.experimental import pallas as pl
from jax.experimental.pallas import tpu as pltpu
from jax.experimental.pallas import tpu_sc as plsc

NC = 2
NS = 16
L = 16
K = 128
RB = 2048


@functools.cache
def _degree_kernel(npad: int, epad: int):
    ed = epad // (NC * NS)

    @functools.partial(
        pl.kernel,
        out_type=jax.ShapeDtypeStruct((2, NC * NS, npad), jnp.float32),
        mesh=plsc.VectorSubcoreMesh(core_axis_name="c", subcore_axis_name="s"),
        compiler_params=pltpu.CompilerParams(needs_layout_passes=False),
        scratch_types=[
            pltpu.VMEM((ed,), jnp.int32),
            pltpu.VMEM((ed,), jnp.int32),
            pltpu.VMEM((npad,), jnp.float32),
            pltpu.VMEM((npad,), jnp.float32),
        ],
    )
    def deg(src_hbm, dst_hbm, out_hbm, src_v, dst_v, hsrc, hdst):
        c = lax.axis_index("c")
        s = lax.axis_index("s")
        w = s * NC + c
        pltpu.sync_copy(src_hbm.at[pl.ds(w * ed, ed)], src_v)
        pltpu.sync_copy(dst_hbm.at[pl.ds(w * ed, ed)], dst_v)

        zero = jnp.zeros((L,), jnp.float32)

        def zbody(i, _):
            hsrc[pl.ds(i * L, L)] = zero
            hdst[pl.ds(i * L, L)] = zero
            return 0

        lax.fori_loop(0, npad // L, zbody, 0)

        ones = jnp.full((L,), 1.0, jnp.float32)

        def ebody(i, _):
            plsc.addupdate_scatter(hsrc, [src_v[pl.ds(i * L, L)]], ones)
            plsc.addupdate_scatter(hdst, [dst_v[pl.ds(i * L, L)]], ones)
            return 0

        lax.fori_loop(0, ed // L, ebody, 0)

        pltpu.sync_copy(hsrc, out_hbm.at[0, w])
        pltpu.sync_copy(hdst, out_hbm.at[1, w])

    return deg


@functools.cache
def _edge_pass_kernel(npad: int, f: int, ech: int, feature_split: bool):
    rpw = npad // NS
    sch = 16
    assert ech % sch == 0

    @functools.partial(
        pl.kernel,
        out_type=jax.ShapeDtypeStruct((NC, npad, f), jnp.float32),
        mesh=plsc.VectorSubcoreMesh(core_axis_name="c", subcore_axis_name="s"),
        compiler_params=pltpu.CompilerParams(needs_layout_passes=False),
        scratch_types=[
            pltpu.VMEM((sch * K,), jnp.int32),
            pltpu.VMEM((sch, K), jnp.int32),
            pltpu.VMEM((K, f), jnp.float32),
            pltpu.VMEM((K, f), jnp.float32),
            pltpu.VMEM_SHARED((npad, f), jnp.float32),
            pltpu.SemaphoreType.DMA,
            pltpu.SemaphoreType.DMA,
        ],
    )
    def ep(tbl_hbm, srcb_hbm, dst2_hbm, out_hbm,
           src_v, dst_v, buf0, buf1, acc, gsem, ssem):
        c = lax.axis_index("c")
        s = lax.axis_index("s")
        if feature_split:
            row, cbase = c, s * ech
        else:
            row, cbase = 0, (s * NC + c) * ech

        zero = jnp.zeros((L,), jnp.float32)

        def zbody(i, _):
            for k in range(f // L):
                buf0[i, pl.ds(k * L, L)] = zero
            return 0

        lax.fori_loop(0, K, zbody, 0)
        for t in range(rpw // K):
            pltpu.sync_copy(buf0, acc.at[pl.ds(s * rpw + t * K, K)])
        plsc.subcore_barrier()

        def g_start(j, buf):
            pltpu.async_copy(tbl_hbm.at[src_v.at[pl.ds(j * K, K)]], buf, gsem)

        def g_wait(j, buf):
            pltpu.make_async_copy(
                tbl_hbm.at[src_v.at[pl.ds(j * K, K)]], buf, gsem).wait()

        def s_start(j, buf):
            pltpu.async_copy(buf, acc.at[dst_v.at[j]], ssem, add=True)

        def s_wait(j, buf):
            pltpu.make_async_copy(buf, acc.at[dst_v.at[j]], ssem).wait()

        def superchunk(m, _):
            base = cbase + m * sch
            pltpu.sync_copy(srcb_hbm.at[row, pl.ds(base * K, sch * K)], src_v)
            pltpu.sync_copy(dst2_hbm.at[pl.ds(base, sch)], dst_v)
            g_start(0, buf0)
            g_wait(0, buf0)
            s_start(0, buf0)
            g_start(1, buf1)
            for p in range((sch - 2) // 2):
                j1 = 2 * p + 1
                g_wait(j1, buf1)
                s_start(j1, buf1)
                s_wait(j1 - 1, buf0)
                g_start(j1 + 1, buf0)
                j2 = 2 * p + 2
                g_wait(j2, buf0)
                s_start(j2, buf0)
                s_wait(j2 - 1, buf1)
                g_start(j2 + 1, buf1)
            jl = sch - 1
            g_wait(jl, buf1)
            s_start(jl, buf1)
            s_wait(jl - 1, buf0)
            s_wait(jl, buf1)
            return 0

        lax.fori_loop(0, ech // sch, superchunk, 0)

        plsc.subcore_barrier()
        pltpu.sync_copy(acc.at[pl.ds(s * rpw, rpw)],
                        out_hbm.at[c, pl.ds(s * rpw, rpw)])

    return ep


def _run_degree(src_p, dst_p, npad, epad):
    return _degree_kernel(npad, epad)(src_p, dst_p)


def _run_edge_pass(tbl_flat, srcb, dst2, npad, f, ech, feature_split):
    return _edge_pass_kernel(npad, f, ech, feature_split)(tbl_flat, srcb, dst2)


def _prep_body(n, degp_ref, x_ref, xs_ref, ns_ref, nd_ref):
    d = jnp.sum(degp_ref[...], axis=1)
    rows = jax.lax.broadcasted_iota(jnp.int32, (RB,), 0) + pl.program_id(0) * RB
    ns = jnp.where((d[0] > 0) & (rows < n), lax.rsqrt(d[0]), 0.0)
    nd = jnp.where(d[1] > 0, lax.rsqrt(d[1]), 0.0)
    ns_ref[...] = ns[:, None]
    nd_ref[...] = nd[:, None]
    xs_ref[...] = x_ref[...] * ns[:, None]


def _run_prep(degp, xp, n, npad, fin):
    nw = degp.shape[1]
    return pl.pallas_call(
        functools.partial(_prep_body, n),
        grid=(npad // RB,),
        in_specs=[
            pl.BlockSpec((2, nw, RB), lambda i: (0, 0, i)),
            pl.BlockSpec((RB, fin), lambda i: (i, 0)),
        ],
        out_specs=[
            pl.BlockSpec((RB, fin), lambda i: (i, 0)),
            pl.BlockSpec((RB, 1), lambda i: (i, 0)),
            pl.BlockSpec((RB, 1), lambda i: (i, 0)),
        ],
        out_shape=[
            jax.ShapeDtypeStruct((npad, fin), jnp.float32),
            jax.ShapeDtypeStruct((npad, 1), jnp.float32),
            jax.ShapeDtypeStruct((npad, 1), jnp.float32),
        ],
    )(degp, xp)


def _l1_body(a_ref, w_ref, b_ref, nd_ref, ns_ref, hs_ref):
    a = a_ref[0] + a_ref[1]
    acc = jnp.dot(a, w_ref[...], preferred_element_type=jnp.float32)
    h = acc * nd_ref[...] + b_ref[...]
    h = jnp.maximum(h, 0.0) * ns_ref[...]
    hh = h.shape[1] // 2
    hs_ref[0] = h[:, :hh]
    hs_ref[1] = h[:, hh:]


def _run_l1(agg, w1, b1, nd, ns, npad, fin, hid):
    return pl.pallas_call(
        _l1_body,
        grid=(npad // RB,),
        in_specs=[
            pl.BlockSpec((2, RB, fin), lambda i: (0, i, 0)),
            pl.BlockSpec((fin, hid), lambda i: (0, 0)),
            pl.BlockSpec((1, hid), lambda i: (0, 0)),
            pl.BlockSpec((RB, 1), lambda i: (i, 0)),
            pl.BlockSpec((RB, 1), lambda i: (i, 0)),
        ],
        out_specs=pl.BlockSpec((2, RB, hid // 2), lambda i: (0, i, 0)),
        out_shape=jax.ShapeDtypeStruct((2, npad, hid // 2), jnp.float32),
    )(agg, w1, b1, nd, ns)


def _l2_body(a_ref, w_ref, b_ref, nd_ref, out_ref):
    w = w_ref[...]
    half = a_ref.shape[2]
    acc = (jnp.dot(a_ref[0], w[:half], preferred_element_type=jnp.float32)
           + jnp.dot(a_ref[1], w[half:], preferred_element_type=jnp.float32))
    out_ref[...] = acc * nd_ref[...] + b_ref[...]


def _run_l2(agg, w2, b2, nd, npad, hid):
    return pl.pallas_call(
        _l2_body,
        grid=(npad // RB,),
        in_specs=[
            pl.BlockSpec((2, RB, hid // 2), lambda i: (0, i, 0)),
            pl.BlockSpec((hid, hid), lambda i: (0, 0)),
            pl.BlockSpec((1, hid), lambda i: (0, 0)),
            pl.BlockSpec((RB, 1), lambda i: (i, 0)),
        ],
        out_specs=pl.BlockSpec((RB, hid), lambda i: (i, 0)),
        out_shape=jax.ShapeDtypeStruct((npad, hid), jnp.float32),
    )(agg, w2, b2, nd)


def kernel(x, edge_index, W1, b1, W2, b2):
    n, fin = x.shape
    hid = W1.shape[1]
    e = edge_index.shape[1]

    npad = -(-n // RB) * RB
    if npad == n:
        npad += RB
    grp = 8 * K * NS * NC
    epad = -(-e // grp) * grp
    ech1 = epad // (K * NS * NC)
    ech2 = epad // (K * NS)

    src = edge_index[0].astype(jnp.int32)
    dst = edge_index[1].astype(jnp.int32)
    fill = n + (jnp.arange(epad - e, dtype=jnp.int32) % (npad - n))
    src_p = jnp.concatenate([src, fill])
    dst_p = jnp.concatenate([dst, fill])
    srcb = jnp.stack([src_p, src_p + npad])
    dst2 = dst_p.reshape(epad // K, K)
    xp = jnp.pad(x, ((0, npad - n), (0, 0)))

    degp = _run_degree(src_p, dst_p, npad, epad)
    xs, ns, nd = _run_prep(degp, xp, n, npad, fin)
    agg1 = _run_edge_pass(xs, srcb, dst2, npad, fin, ech1, False)
    hs = _run_l1(agg1, W1, b1.reshape(1, hid), nd, ns, npad, fin, hid)
    agg2 = _run_edge_pass(hs.reshape(NC * npad, hid // 2), srcb, dst2,
                          npad, hid // 2, ech2, True)
    out = _run_l2(agg2, W2, b2.reshape(1, hid), nd, npad, hid)
    return out[:n]


# speedup vs baseline: 10.2683x; 1.0437x over previous
import functools

import jax
import jax.numpy as jnp
from jax import lax
from jax.experimental import pallas as pl
from jax.experimental.pallas import tpu as pltpu
from jax.experimental.pallas import tpu_sc as plsc

NC = 2
NS = 16
L = 16
K = 128
RB = 2048


@functools.cache
def _degree_kernel(npad: int, epad: int):
    ed = epad // (NC * NS)

    @functools.partial(
        pl.kernel,
        out_type=jax.ShapeDtypeStruct((2, NC * NS, npad), jnp.float32),
        mesh=plsc.VectorSubcoreMesh(core_axis_name="c", subcore_axis_name="s"),
        compiler_params=pltpu.CompilerParams(needs_layout_passes=False),
        scratch_types=[
            pltpu.VMEM((ed,), jnp.int32),
            pltpu.VMEM((ed,), jnp.int32),
            pltpu.VMEM((npad,), jnp.float32),
            pltpu.VMEM((npad,), jnp.float32),
        ],
    )
    def deg(src_hbm, dst_hbm, out_hbm, src_v, dst_v, hsrc, hdst):
        c = lax.axis_index("c")
        s = lax.axis_index("s")
        w = s * NC + c
        pltpu.sync_copy(src_hbm.at[pl.ds(w * ed, ed)], src_v)
        pltpu.sync_copy(dst_hbm.at[pl.ds(w * ed, ed)], dst_v)

        zero = jnp.zeros((L,), jnp.float32)

        def zbody(i, _):
            hsrc[pl.ds(i * L, L)] = zero
            hdst[pl.ds(i * L, L)] = zero
            return 0

        lax.fori_loop(0, npad // L, zbody, 0)

        ones = jnp.full((L,), 1.0, jnp.float32)

        def ebody(i, _):
            plsc.addupdate_scatter(hsrc, [src_v[pl.ds(i * L, L)]], ones)
            plsc.addupdate_scatter(hdst, [dst_v[pl.ds(i * L, L)]], ones)
            return 0

        lax.fori_loop(0, ed // L, ebody, 0)

        pltpu.sync_copy(hsrc, out_hbm.at[0, w])
        pltpu.sync_copy(hdst, out_hbm.at[1, w])

    return deg


@functools.cache
def _edge_pass_kernel(npad: int, f: int, ech: int, feature_split: bool):
    rpw = npad // NS
    sch = 8
    nsc = ech // sch
    assert ech % sch == 0 and nsc % 2 == 0 and nsc >= 4

    @functools.partial(
        pl.kernel,
        out_type=jax.ShapeDtypeStruct((NC, npad, f), jnp.float32),
        mesh=plsc.VectorSubcoreMesh(core_axis_name="c", subcore_axis_name="s"),
        compiler_params=pltpu.CompilerParams(needs_layout_passes=False),
        scratch_types=[
            pltpu.VMEM((sch * K,), jnp.int32),
            pltpu.VMEM((sch, K), jnp.int32),
            pltpu.VMEM((sch * K,), jnp.int32),
            pltpu.VMEM((sch, K), jnp.int32),
            pltpu.VMEM((K, f), jnp.float32),
            pltpu.VMEM((K, f), jnp.float32),
            pltpu.VMEM_SHARED((npad, f), jnp.float32),
            pltpu.SemaphoreType.DMA,
            pltpu.SemaphoreType.DMA,
            pltpu.SemaphoreType.DMA,
        ],
    )
    def ep(tbl_hbm, srcb_hbm, dst2_hbm, out_hbm,
           src_a, dst_a, src_b, dst_b, buf0, buf1, acc, gsem, ssem, isem):
        c = lax.axis_index("c")
        s = lax.axis_index("s")
        if feature_split:
            row, cbase = c, s * ech
        else:
            row, cbase = 0, (s * NC + c) * ech

        idx = [(src_a, dst_a), (src_b, dst_b)]
        bufs = [buf0, buf1]

        zero = jnp.zeros((L,), jnp.float32)

        def zbody(i, _):
            for k in range(f // L):
                buf0[i, pl.ds(k * L, L)] = zero
            return 0

        lax.fori_loop(0, K, zbody, 0)
        for t in range(rpw // K):
            pltpu.sync_copy(buf0, acc.at[pl.ds(s * rpw + t * K, K)])
        plsc.subcore_barrier()

        def i_start(m, p):
            sv, dv = idx[p]
            base = cbase + m * sch
            pltpu.async_copy(srcb_hbm.at[row, pl.ds(base * K, sch * K)],
                             sv, isem)
            pltpu.async_copy(dst2_hbm.at[pl.ds(base, sch)], dv, isem)

        def i_wait(m, p):
            sv, dv = idx[p]
            base = cbase + m * sch
            pltpu.make_async_copy(
                srcb_hbm.at[row, pl.ds(base * K, sch * K)], sv, isem).wait()
            pltpu.make_async_copy(
                dst2_hbm.at[pl.ds(base, sch)], dv, isem).wait()

        def g_start(jj, p, bp):
            sv, _ = idx[p]
            pltpu.async_copy(
                tbl_hbm.at[sv.at[pl.ds(jj * K, K)]], bufs[bp], gsem)

        def g_wait(jj, p, bp):
            sv, _ = idx[p]
            pltpu.make_async_copy(
                tbl_hbm.at[sv.at[pl.ds(jj * K, K)]], bufs[bp], gsem).wait()

        def s_start(jj, p, bp):
            _, dv = idx[p]
            pltpu.async_copy(bufs[bp], acc.at[dv.at[jj]], ssem, add=True)

        def s_wait(jj, p, bp):
            _, dv = idx[p]
            pltpu.make_async_copy(bufs[bp], acc.at[dv.at[jj]], ssem).wait()

        def steady(jj, p, np_, pw):
            bp = jj % 2
            g_wait(jj, p, bp)
            s_start(jj, p, bp)
            s_wait(pw[0], pw[1], 1 - bp)
            g_start(np_[0], np_[1], 1 - bp)

        i_start(0, 0)
        i_wait(0, 0)
        i_start(1, 1)
        g_start(0, 0, 0)
        g_wait(0, 0, 0)
        s_start(0, 0, 0)
        g_start(1, 0, 1)
        for jj in range(1, sch - 1):
            steady(jj, 0, (jj + 1, 0), (jj - 1, 0))
        i_wait(1, 1)
        steady(sch - 1, 0, (0, 1), (sch - 2, 0))

        def spair(q, _):
            for (m, p) in ((2 * q + 1, 1), (2 * q + 2, 0)):
                bp0 = 0
                g_wait(0, p, bp0)
                s_start(0, p, bp0)
                s_wait(sch - 1, 1 - p, 1 - bp0)
                i_start(m + 1, 1 - p)
                g_start(1, p, 1 - bp0)
                for jj in range(1, sch - 1):
                    steady(jj, p, (jj + 1, p), (jj - 1, p))
                i_wait(m + 1, 1 - p)
                steady(sch - 1, p, (0, 1 - p), (sch - 2, p))
            return 0

        lax.fori_loop(0, (nsc - 2) // 2, spair, 0)

        p = 1
        g_wait(0, p, 0)
        s_start(0, p, 0)
        s_wait(sch - 1, 0, 1)
        g_start(1, p, 1)
        for jj in range(1, sch - 1):
            steady(jj, p, (jj + 1, p), (jj - 1, p))
        jl = sch - 1
        g_wait(jl, p, jl % 2)
        s_start(jl, p, jl % 2)
        s_wait(jl - 1, p, 1 - jl % 2)
        s_wait(jl, p, jl % 2)

        plsc.subcore_barrier()
        pltpu.sync_copy(acc.at[pl.ds(s * rpw, rpw)],
                        out_hbm.at[c, pl.ds(s * rpw, rpw)])

    return ep


def _run_degree(src_p, dst_p, npad, epad):
    return _degree_kernel(npad, epad)(src_p, dst_p)


def _run_edge_pass(tbl_flat, srcb, dst2, npad, f, ech, feature_split):
    return _edge_pass_kernel(npad, f, ech, feature_split)(tbl_flat, srcb, dst2)


def _prep_body(n, degp_ref, x_ref, xs_ref, ns_ref, nd_ref):
    d = jnp.sum(degp_ref[...], axis=1)
    rows = jax.lax.broadcasted_iota(jnp.int32, (RB,), 0) + pl.program_id(0) * RB
    ns = jnp.where((d[0] > 0) & (rows < n), lax.rsqrt(d[0]), 0.0)
    nd = jnp.where(d[1] > 0, lax.rsqrt(d[1]), 0.0)
    ns_ref[...] = ns[:, None]
    nd_ref[...] = nd[:, None]
    xs_ref[...] = x_ref[...] * ns[:, None]


def _run_prep(degp, xp, n, npad, fin):
    nw = degp.shape[1]
    return pl.pallas_call(
        functools.partial(_prep_body, n),
        grid=(npad // RB,),
        in_specs=[
            pl.BlockSpec((2, nw, RB), lambda i: (0, 0, i)),
            pl.BlockSpec((RB, fin), lambda i: (i, 0)),
        ],
        out_specs=[
            pl.BlockSpec((RB, fin), lambda i: (i, 0)),
            pl.BlockSpec((RB, 1), lambda i: (i, 0)),
            pl.BlockSpec((RB, 1), lambda i: (i, 0)),
        ],
        out_shape=[
            jax.ShapeDtypeStruct((npad, fin), jnp.float32),
            jax.ShapeDtypeStruct((npad, 1), jnp.float32),
            jax.ShapeDtypeStruct((npad, 1), jnp.float32),
        ],
    )(degp, xp)


def _l1_body(a_ref, w_ref, b_ref, nd_ref, ns_ref, hs_ref):
    a = a_ref[0] + a_ref[1]
    acc = jnp.dot(a, w_ref[...], preferred_element_type=jnp.float32)
    h = acc * nd_ref[...] + b_ref[...]
    h = jnp.maximum(h, 0.0) * ns_ref[...]
    hh = h.shape[1] // 2
    hs_ref[0] = h[:, :hh]
    hs_ref[1] = h[:, hh:]


def _run_l1(agg, w1, b1, nd, ns, npad, fin, hid):
    return pl.pallas_call(
        _l1_body,
        grid=(npad // RB,),
        in_specs=[
            pl.BlockSpec((2, RB, fin), lambda i: (0, i, 0)),
            pl.BlockSpec((fin, hid), lambda i: (0, 0)),
            pl.BlockSpec((1, hid), lambda i: (0, 0)),
            pl.BlockSpec((RB, 1), lambda i: (i, 0)),
            pl.BlockSpec((RB, 1), lambda i: (i, 0)),
        ],
        out_specs=pl.BlockSpec((2, RB, hid // 2), lambda i: (0, i, 0)),
        out_shape=jax.ShapeDtypeStruct((2, npad, hid // 2), jnp.float32),
    )(agg, w1, b1, nd, ns)


def _l2_body(a_ref, w_ref, b_ref, nd_ref, out_ref):
    w = w_ref[...]
    half = a_ref.shape[2]
    acc = (jnp.dot(a_ref[0], w[:half], preferred_element_type=jnp.float32)
           + jnp.dot(a_ref[1], w[half:], preferred_element_type=jnp.float32))
    out_ref[...] = acc * nd_ref[...] + b_ref[...]


def _run_l2(agg, w2, b2, nd, npad, hid):
    return pl.pallas_call(
        _l2_body,
        grid=(npad // RB,),
        in_specs=[
            pl.BlockSpec((2, RB, hid // 2), lambda i: (0, i, 0)),
            pl.BlockSpec((hid, hid), lambda i: (0, 0)),
            pl.BlockSpec((1, hid), lambda i: (0, 0)),
            pl.BlockSpec((RB, 1), lambda i: (i, 0)),
        ],
        out_specs=pl.BlockSpec((RB, hid), lambda i: (i, 0)),
        out_shape=jax.ShapeDtypeStruct((npad, hid), jnp.float32),
    )(agg, w2, b2, nd)


def kernel(x, edge_index, W1, b1, W2, b2):
    n, fin = x.shape
    hid = W1.shape[1]
    e = edge_index.shape[1]

    npad = -(-n // RB) * RB
    if npad == n:
        npad += RB
    grp = 16 * K * NS * NC
    epad = -(-e // grp) * grp
    ech1 = epad // (K * NS * NC)
    ech2 = epad // (K * NS)

    src = edge_index[0].astype(jnp.int32)
    dst = edge_index[1].astype(jnp.int32)
    fill = n + (jnp.arange(epad - e, dtype=jnp.int32) % (npad - n))
    src_p = jnp.concatenate([src, fill])
    dst_p = jnp.concatenate([dst, fill])
    srcb = jnp.stack([src_p, src_p + npad])
    dst2 = dst_p.reshape(epad // K, K)
    xp = jnp.pad(x, ((0, npad - n), (0, 0)))

    degp = _run_degree(src_p, dst_p, npad, epad)
    xs, ns, nd = _run_prep(degp, xp, n, npad, fin)
    agg1 = _run_edge_pass(xs, srcb, dst2, npad, fin, ech1, False)
    hs = _run_l1(agg1, W1, b1.reshape(1, hid), nd, ns, npad, fin, hid)
    agg2 = _run_edge_pass(hs.reshape(NC * npad, hid // 2), srcb, dst2,
                          npad, hid // 2, ech2, True)
    out = _run_l2(agg2, W2, b2.reshape(1, hid), nd, npad, hid)
    return out[:n]


# speedup vs baseline: 10.3041x; 1.0035x over previous
import functools

import jax
import jax.numpy as jnp
from jax import lax
from jax.experimental import pallas as pl
from jax.experimental.pallas import tpu as pltpu
from jax.experimental.pallas import tpu_sc as plsc

NC = 2
NS = 16
L = 16
K = 128
RB = 2048


@functools.cache
def _degree_kernel(npad: int, epad: int):
    ed = epad // (NC * NS)

    @functools.partial(
        pl.kernel,
        out_type=jax.ShapeDtypeStruct((2, NC * NS, npad), jnp.float32),
        mesh=plsc.VectorSubcoreMesh(core_axis_name="c", subcore_axis_name="s"),
        compiler_params=pltpu.CompilerParams(needs_layout_passes=False),
        scratch_types=[
            pltpu.VMEM((ed,), jnp.int32),
            pltpu.VMEM((ed,), jnp.int32),
            pltpu.VMEM((npad,), jnp.float32),
            pltpu.VMEM((npad,), jnp.float32),
        ],
    )
    def deg(src_hbm, dst_hbm, out_hbm, src_v, dst_v, hsrc, hdst):
        c = lax.axis_index("c")
        s = lax.axis_index("s")
        w = s * NC + c
        pltpu.sync_copy(src_hbm.at[pl.ds(w * ed, ed)], src_v)
        pltpu.sync_copy(dst_hbm.at[pl.ds(w * ed, ed)], dst_v)

        zero = jnp.zeros((L,), jnp.float32)

        def zbody(i, _):
            hsrc[pl.ds(i * L, L)] = zero
            hdst[pl.ds(i * L, L)] = zero
            return 0

        lax.fori_loop(0, npad // L, zbody, 0)

        ones = jnp.full((L,), 1.0, jnp.float32)

        def ebody(i, _):
            plsc.addupdate_scatter(hsrc, [src_v[pl.ds(i * L, L)]], ones)
            plsc.addupdate_scatter(hdst, [dst_v[pl.ds(i * L, L)]], ones)
            return 0

        lax.fori_loop(0, ed // L, ebody, 0)

        pltpu.sync_copy(hsrc, out_hbm.at[0, w])
        pltpu.sync_copy(hdst, out_hbm.at[1, w])

    return deg


@functools.cache
def _edge_pass_kernel(npad: int, f: int, ech: int, feature_split: bool):
    rpw = npad // NS
    sch = 8
    nsc = ech // sch
    assert ech % sch == 0 and nsc % 2 == 0 and nsc >= 4

    @functools.partial(
        pl.kernel,
        out_type=jax.ShapeDtypeStruct((NC, npad, f), jnp.float32),
        mesh=plsc.VectorSubcoreMesh(core_axis_name="c", subcore_axis_name="s"),
        compiler_params=pltpu.CompilerParams(needs_layout_passes=False),
        scratch_types=[
            pltpu.VMEM((sch * K,), jnp.int32),
            pltpu.VMEM((sch, K), jnp.int32),
            pltpu.VMEM((sch * K,), jnp.int32),
            pltpu.VMEM((sch, K), jnp.int32),
            pltpu.VMEM((K, f), jnp.float32),
            pltpu.VMEM((K, f), jnp.float32),
            pltpu.VMEM_SHARED((npad, f), jnp.float32),
            pltpu.SemaphoreType.DMA,
            pltpu.SemaphoreType.DMA,
            pltpu.SemaphoreType.DMA,
        ],
    )
    def ep(tbl_hbm, srcb_hbm, dst2_hbm, out_hbm,
           src_a, dst_a, src_b, dst_b, buf0, buf1, acc, gsem, ssem, isem):
        c = lax.axis_index("c")
        s = lax.axis_index("s")
        if feature_split:
            row, cbase = c, s * ech
        else:
            row, cbase = 0, (s * NC + c) * ech

        idx = [(src_a, dst_a), (src_b, dst_b)]
        bufs = [buf0, buf1]

        zero = jnp.zeros((L,), jnp.float32)

        def zbody(i, _):
            for k in range(f // L):
                buf0[i, pl.ds(k * L, L)] = zero
            return 0

        lax.fori_loop(0, K, zbody, 0)
        for t in range(rpw // K):
            pltpu.sync_copy(buf0, acc.at[pl.ds(s * rpw + t * K, K)])
        plsc.subcore_barrier()

        def i_start(m, p):
            sv, dv = idx[p]
            base = cbase + m * sch
            pltpu.async_copy(srcb_hbm.at[row, pl.ds(base * K, sch * K)],
                             sv, isem)
            pltpu.async_copy(dst2_hbm.at[pl.ds(base, sch)], dv, isem)

        def i_wait(m, p):
            sv, dv = idx[p]
            base = cbase + m * sch
            pltpu.make_async_copy(
                srcb_hbm.at[row, pl.ds(base * K, sch * K)], sv, isem).wait()
            pltpu.make_async_copy(
                dst2_hbm.at[pl.ds(base, sch)], dv, isem).wait()

        def g_start(jj, p, bp):
            sv, _ = idx[p]
            pltpu.async_copy(
                tbl_hbm.at[sv.at[pl.ds(jj * K, K)]], bufs[bp], gsem)

        def g_wait(jj, p, bp):
            sv, _ = idx[p]
            pltpu.make_async_copy(
                tbl_hbm.at[sv.at[pl.ds(jj * K, K)]], bufs[bp], gsem).wait()

        def s_start(jj, p, bp):
            _, dv = idx[p]
            pltpu.async_copy(bufs[bp], acc.at[dv.at[jj]], ssem, add=True)

        def s_wait(jj, p, bp):
            _, dv = idx[p]
            pltpu.make_async_copy(bufs[bp], acc.at[dv.at[jj]], ssem).wait()

        def steady(jj, p, np_, pw):
            bp = jj % 2
            g_wait(jj, p, bp)
            s_wait(pw[0], pw[1], 1 - bp)
            g_start(np_[0], np_[1], 1 - bp)
            s_start(jj, p, bp)

        i_start(0, 0)
        i_wait(0, 0)
        i_start(1, 1)
        g_start(0, 0, 0)
        g_start(1, 0, 1)
        g_wait(0, 0, 0)
        s_start(0, 0, 0)
        for jj in range(1, sch - 1):
            steady(jj, 0, (jj + 1, 0), (jj - 1, 0))
        i_wait(1, 1)
        steady(sch - 1, 0, (0, 1), (sch - 2, 0))

        def spair(q, _):
            for (m, p) in ((2 * q + 1, 1), (2 * q + 2, 0)):
                bp0 = 0
                g_wait(0, p, bp0)
                s_wait(sch - 1, 1 - p, 1 - bp0)
                g_start(1, p, 1 - bp0)
                s_start(0, p, bp0)
                i_start(m + 1, 1 - p)
                for jj in range(1, sch - 1):
                    steady(jj, p, (jj + 1, p), (jj - 1, p))
                i_wait(m + 1, 1 - p)
                steady(sch - 1, p, (0, 1 - p), (sch - 2, p))
            return 0

        lax.fori_loop(0, (nsc - 2) // 2, spair, 0)

        p = 1
        g_wait(0, p, 0)
        s_wait(sch - 1, 0, 1)
        g_start(1, p, 1)
        s_start(0, p, 0)
        for jj in range(1, sch - 1):
            steady(jj, p, (jj + 1, p), (jj - 1, p))
        jl = sch - 1
        g_wait(jl, p, jl % 2)
        s_start(jl, p, jl % 2)
        s_wait(jl - 1, p, 1 - jl % 2)
        s_wait(jl, p, jl % 2)

        plsc.subcore_barrier()
        pltpu.sync_copy(acc.at[pl.ds(s * rpw, rpw)],
                        out_hbm.at[c, pl.ds(s * rpw, rpw)])

    return ep


def _run_degree(src_p, dst_p, npad, epad):
    return _degree_kernel(npad, epad)(src_p, dst_p)


def _run_edge_pass(tbl_flat, srcb, dst2, npad, f, ech, feature_split):
    return _edge_pass_kernel(npad, f, ech, feature_split)(tbl_flat, srcb, dst2)


def _prep_body(n, degp_ref, x_ref, xs_ref, ns_ref, nd_ref):
    d = jnp.sum(degp_ref[...], axis=1)
    rows = jax.lax.broadcasted_iota(jnp.int32, (RB,), 0) + pl.program_id(0) * RB
    ns = jnp.where((d[0] > 0) & (rows < n), lax.rsqrt(d[0]), 0.0)
    nd = jnp.where(d[1] > 0, lax.rsqrt(d[1]), 0.0)
    ns_ref[...] = ns[:, None]
    nd_ref[...] = nd[:, None]
    xs_ref[...] = x_ref[...] * ns[:, None]


def _run_prep(degp, xp, n, npad, fin):
    nw = degp.shape[1]
    return pl.pallas_call(
        functools.partial(_prep_body, n),
        grid=(npad // RB,),
        in_specs=[
            pl.BlockSpec((2, nw, RB), lambda i: (0, 0, i)),
            pl.BlockSpec((RB, fin), lambda i: (i, 0)),
        ],
        out_specs=[
            pl.BlockSpec((RB, fin), lambda i: (i, 0)),
            pl.BlockSpec((RB, 1), lambda i: (i, 0)),
            pl.BlockSpec((RB, 1), lambda i: (i, 0)),
        ],
        out_shape=[
            jax.ShapeDtypeStruct((npad, fin), jnp.float32),
            jax.ShapeDtypeStruct((npad, 1), jnp.float32),
            jax.ShapeDtypeStruct((npad, 1), jnp.float32),
        ],
    )(degp, xp)


def _l1_body(a_ref, w_ref, b_ref, nd_ref, ns_ref, hs_ref):
    a = a_ref[0] + a_ref[1]
    acc = jnp.dot(a, w_ref[...], preferred_element_type=jnp.float32)
    h = acc * nd_ref[...] + b_ref[...]
    h = jnp.maximum(h, 0.0) * ns_ref[...]
    hh = h.shape[1] // 2
    hs_ref[0] = h[:, :hh]
    hs_ref[1] = h[:, hh:]


def _run_l1(agg, w1, b1, nd, ns, npad, fin, hid):
    return pl.pallas_call(
        _l1_body,
        grid=(npad // RB,),
        in_specs=[
            pl.BlockSpec((2, RB, fin), lambda i: (0, i, 0)),
            pl.BlockSpec((fin, hid), lambda i: (0, 0)),
            pl.BlockSpec((1, hid), lambda i: (0, 0)),
            pl.BlockSpec((RB, 1), lambda i: (i, 0)),
            pl.BlockSpec((RB, 1), lambda i: (i, 0)),
        ],
        out_specs=pl.BlockSpec((2, RB, hid // 2), lambda i: (0, i, 0)),
        out_shape=jax.ShapeDtypeStruct((2, npad, hid // 2), jnp.float32),
    )(agg, w1, b1, nd, ns)


def _l2_body(a_ref, w_ref, b_ref, nd_ref, out_ref):
    w = w_ref[...]
    half = a_ref.shape[2]
    acc = (jnp.dot(a_ref[0], w[:half], preferred_element_type=jnp.float32)
           + jnp.dot(a_ref[1], w[half:], preferred_element_type=jnp.float32))
    out_ref[...] = acc * nd_ref[...] + b_ref[...]


def _run_l2(agg, w2, b2, nd, npad, hid):
    return pl.pallas_call(
        _l2_body,
        grid=(npad // RB,),
        in_specs=[
            pl.BlockSpec((2, RB, hid // 2), lambda i: (0, i, 0)),
            pl.BlockSpec((hid, hid), lambda i: (0, 0)),
            pl.BlockSpec((1, hid), lambda i: (0, 0)),
            pl.BlockSpec((RB, 1), lambda i: (i, 0)),
        ],
        out_specs=pl.BlockSpec((RB, hid), lambda i: (i, 0)),
        out_shape=jax.ShapeDtypeStruct((npad, hid), jnp.float32),
    )(agg, w2, b2, nd)


def kernel(x, edge_index, W1, b1, W2, b2):
    n, fin = x.shape
    hid = W1.shape[1]
    e = edge_index.shape[1]

    npad = -(-n // RB) * RB
    if npad == n:
        npad += RB
    grp = 16 * K * NS * NC
    epad = -(-e // grp) * grp
    ech1 = epad // (K * NS * NC)
    ech2 = epad // (K * NS)

    src = edge_index[0].astype(jnp.int32)
    dst = edge_index[1].astype(jnp.int32)
    fill = n + (jnp.arange(epad - e, dtype=jnp.int32) % (npad - n))
    src_p = jnp.concatenate([src, fill])
    dst_p = jnp.concatenate([dst, fill])
    srcb = jnp.stack([src_p, src_p + npad])
    dst2 = dst_p.reshape(epad // K, K)
    xp = jnp.pad(x, ((0, npad - n), (0, 0)))

    degp = _run_degree(src_p, dst_p, npad, epad)
    xs, ns, nd = _run_prep(degp, xp, n, npad, fin)
    agg1 = _run_edge_pass(xs, srcb, dst2, npad, fin, ech1, False)
    hs = _run_l1(agg1, W1, b1.reshape(1, hid), nd, ns, npad, fin, hid)
    agg2 = _run_edge_pass(hs.reshape(NC * npad, hid // 2), srcb, dst2,
                          npad, hid // 2, ech2, True)
    out = _run_l2(agg2, W2, b2.reshape(1, hid), nd, npad, hid)
    return out[:n]


# speedup vs baseline: 10.4070x; 1.0100x over previous
import functools

import jax
import jax.numpy as jnp
from jax import lax
from jax.experimental import pallas as pl
from jax.experimental.pallas import tpu as pltpu
from jax.experimental.pallas import tpu_sc as plsc

NC = 2
NS = 16
L = 16
K = 128
RB = 2048


@functools.cache
def _degree_kernel(npad: int, epad: int):
    ed = epad // (NC * NS)

    @functools.partial(
        pl.kernel,
        out_type=jax.ShapeDtypeStruct((2, NC * NS, npad), jnp.float32),
        mesh=plsc.VectorSubcoreMesh(core_axis_name="c", subcore_axis_name="s"),
        compiler_params=pltpu.CompilerParams(needs_layout_passes=False),
        scratch_types=[
            pltpu.VMEM((ed,), jnp.int32),
            pltpu.VMEM((ed,), jnp.int32),
            pltpu.VMEM((npad,), jnp.float32),
            pltpu.VMEM((npad,), jnp.float32),
        ],
    )
    def deg(src_hbm, dst_hbm, out_hbm, src_v, dst_v, hsrc, hdst):
        c = lax.axis_index("c")
        s = lax.axis_index("s")
        w = s * NC + c
        pltpu.sync_copy(src_hbm.at[pl.ds(w * ed, ed)], src_v)
        pltpu.sync_copy(dst_hbm.at[pl.ds(w * ed, ed)], dst_v)

        zero = jnp.zeros((L,), jnp.float32)

        def zbody(i, _):
            hsrc[pl.ds(i * L, L)] = zero
            hdst[pl.ds(i * L, L)] = zero
            return 0

        lax.fori_loop(0, npad // L, zbody, 0)

        ones = jnp.full((L,), 1.0, jnp.float32)

        def ebody(i, _):
            plsc.addupdate_scatter(hsrc, [src_v[pl.ds(i * L, L)]], ones)
            plsc.addupdate_scatter(hdst, [dst_v[pl.ds(i * L, L)]], ones)
            return 0

        lax.fori_loop(0, ed // L, ebody, 0)

        pltpu.sync_copy(hsrc, out_hbm.at[0, w])
        pltpu.sync_copy(hdst, out_hbm.at[1, w])

    return deg


@functools.cache
def _edge_pass_kernel(npad: int, f: int, ech: int, feature_split: bool):
    rpw = npad // NS
    sch = 8
    nsc = ech // sch
    assert ech % sch == 0 and nsc % 2 == 0 and nsc >= 4

    @functools.partial(
        pl.kernel,
        out_type=jax.ShapeDtypeStruct((NC, npad, f), jnp.float32),
        mesh=plsc.VectorSubcoreMesh(core_axis_name="c", subcore_axis_name="s"),
        compiler_params=pltpu.CompilerParams(needs_layout_passes=False),
        scratch_types=[
            pltpu.VMEM((sch * K,), jnp.int32),
            pltpu.VMEM((sch, K), jnp.int32),
            pltpu.VMEM((sch * K,), jnp.int32),
            pltpu.VMEM((sch, K), jnp.int32),
            pltpu.VMEM((K, f), jnp.float32),
            pltpu.VMEM((K, f), jnp.float32),
            pltpu.VMEM_SHARED((npad, f), jnp.float32),
            pltpu.SemaphoreType.DMA,
            pltpu.SemaphoreType.DMA,
            pltpu.SemaphoreType.DMA,
        ],
    )
    def ep(tbl_hbm, srcb_hbm, dst2_hbm, out_hbm,
           src_a, dst_a, src_b, dst_b, buf0, buf1, acc, gsem, ssem, isem):
        c = lax.axis_index("c")
        s = lax.axis_index("s")
        if feature_split:
            tbl = tbl_hbm.at[pl.ds(c * npad, npad)]
            cbase = s * ech
        else:
            tbl = tbl_hbm
            cbase = (s * NC + c) * ech

        idx = [(src_a, dst_a), (src_b, dst_b)]
        bufs = [buf0, buf1]

        zero = jnp.zeros((L,), jnp.float32)

        def zbody(i, _):
            for k in range(f // L):
                buf0[i, pl.ds(k * L, L)] = zero
            return 0

        lax.fori_loop(0, K, zbody, 0)
        for t in range(rpw // K):
            pltpu.sync_copy(buf0, acc.at[pl.ds(s * rpw + t * K, K)])
        plsc.subcore_barrier()

        def i_start(m, p):
            sv, dv = idx[p]
            base = cbase + m * sch
            pltpu.async_copy(srcb_hbm.at[pl.ds(base * K, sch * K)], sv, isem)
            pltpu.async_copy(dst2_hbm.at[pl.ds(base, sch)], dv, isem)

        def i_wait(m, p):
            sv, dv = idx[p]
            base = cbase + m * sch
            pltpu.make_async_copy(
                srcb_hbm.at[pl.ds(base * K, sch * K)], sv, isem).wait()
            pltpu.make_async_copy(
                dst2_hbm.at[pl.ds(base, sch)], dv, isem).wait()

        def g_start(jj, p, bp):
            sv, _ = idx[p]
            pltpu.async_copy(
                tbl.at[sv.at[pl.ds(jj * K, K)]], bufs[bp], gsem)

        def g_wait(jj, p, bp):
            sv, _ = idx[p]
            pltpu.make_async_copy(
                tbl.at[sv.at[pl.ds(jj * K, K)]], bufs[bp], gsem).wait()

        def s_start(jj, p, bp):
            _, dv = idx[p]
            pltpu.async_copy(bufs[bp], acc.at[dv.at[jj]], ssem, add=True)

        def s_wait(jj, p, bp):
            _, dv = idx[p]
            pltpu.make_async_copy(bufs[bp], acc.at[dv.at[jj]], ssem).wait()

        def steady(jj, p, np_, pw):
            bp = jj % 2
            g_wait(jj, p, bp)
            s_wait(pw[0], pw[1], 1 - bp)
            g_start(np_[0], np_[1], 1 - bp)
            s_start(jj, p, bp)

        i_start(0, 0)
        i_wait(0, 0)
        i_start(1, 1)
        g_start(0, 0, 0)
        g_start(1, 0, 1)
        g_wait(0, 0, 0)
        s_start(0, 0, 0)
        for jj in range(1, sch - 1):
            steady(jj, 0, (jj + 1, 0), (jj - 1, 0))
        i_wait(1, 1)
        steady(sch - 1, 0, (0, 1), (sch - 2, 0))

        def spair(q, _):
            for (m, p) in ((2 * q + 1, 1), (2 * q + 2, 0)):
                bp0 = 0
                g_wait(0, p, bp0)
                s_wait(sch - 1, 1 - p, 1 - bp0)
                g_start(1, p, 1 - bp0)
                s_start(0, p, bp0)
                i_start(m + 1, 1 - p)
                for jj in range(1, sch - 1):
                    steady(jj, p, (jj + 1, p), (jj - 1, p))
                i_wait(m + 1, 1 - p)
                steady(sch - 1, p, (0, 1 - p), (sch - 2, p))
            return 0

        lax.fori_loop(0, (nsc - 2) // 2, spair, 0)

        p = 1
        g_wait(0, p, 0)
        s_wait(sch - 1, 0, 1)
        g_start(1, p, 1)
        s_start(0, p, 0)
        for jj in range(1, sch - 1):
            steady(jj, p, (jj + 1, p), (jj - 1, p))
        jl = sch - 1
        g_wait(jl, p, jl % 2)
        s_start(jl, p, jl % 2)
        s_wait(jl - 1, p, 1 - jl % 2)
        s_wait(jl, p, jl % 2)

        plsc.subcore_barrier()
        pltpu.sync_copy(acc.at[pl.ds(s * rpw, rpw)],
                        out_hbm.at[c, pl.ds(s * rpw, rpw)])

    return ep


def _run_degree(src_p, dst_p, npad, epad):
    return _degree_kernel(npad, epad)(src_p, dst_p)


def _run_edge_pass(tbl_flat, srcb, dst2, npad, f, ech, feature_split):
    return _edge_pass_kernel(npad, f, ech, feature_split)(tbl_flat, srcb, dst2)


def _prep_body(n, degp_ref, x_ref, xs_ref, ns_ref, nd_ref):
    d = jnp.sum(degp_ref[...], axis=1)
    rows = jax.lax.broadcasted_iota(jnp.int32, (RB,), 0) + pl.program_id(0) * RB
    ns = jnp.where((d[0] > 0) & (rows < n), lax.rsqrt(d[0]), 0.0)
    nd = jnp.where(d[1] > 0, lax.rsqrt(d[1]), 0.0)
    ns_ref[...] = ns[:, None]
    nd_ref[...] = nd[:, None]
    xs_ref[...] = x_ref[...] * ns[:, None]


def _run_prep(degp, xp, n, npad, fin):
    nw = degp.shape[1]
    return pl.pallas_call(
        functools.partial(_prep_body, n),
        grid=(npad // RB,),
        in_specs=[
            pl.BlockSpec((2, nw, RB), lambda i: (0, 0, i)),
            pl.BlockSpec((RB, fin), lambda i: (i, 0)),
        ],
        out_specs=[
            pl.BlockSpec((RB, fin), lambda i: (i, 0)),
            pl.BlockSpec((RB, 1), lambda i: (i, 0)),
            pl.BlockSpec((RB, 1), lambda i: (i, 0)),
        ],
        out_shape=[
            jax.ShapeDtypeStruct((npad, fin), jnp.float32),
            jax.ShapeDtypeStruct((npad, 1), jnp.float32),
            jax.ShapeDtypeStruct((npad, 1), jnp.float32),
        ],
    )(degp, xp)


def _l1_body(a_ref, w_ref, b_ref, nd_ref, ns_ref, hs_ref):
    a = a_ref[0] + a_ref[1]
    acc = jnp.dot(a, w_ref[...], preferred_element_type=jnp.float32)
    h = acc * nd_ref[...] + b_ref[...]
    h = jnp.maximum(h, 0.0) * ns_ref[...]
    hh = h.shape[1] // 2
    hs_ref[0] = h[:, :hh]
    hs_ref[1] = h[:, hh:]


def _run_l1(agg, w1, b1, nd, ns, npad, fin, hid):
    return pl.pallas_call(
        _l1_body,
        grid=(npad // RB,),
        in_specs=[
            pl.BlockSpec((2, RB, fin), lambda i: (0, i, 0)),
            pl.BlockSpec((fin, hid), lambda i: (0, 0)),
            pl.BlockSpec((1, hid), lambda i: (0, 0)),
            pl.BlockSpec((RB, 1), lambda i: (i, 0)),
            pl.BlockSpec((RB, 1), lambda i: (i, 0)),
        ],
        out_specs=pl.BlockSpec((2, RB, hid // 2), lambda i: (0, i, 0)),
        out_shape=jax.ShapeDtypeStruct((2, npad, hid // 2), jnp.float32),
    )(agg, w1, b1, nd, ns)


def _l2_body(a_ref, w_ref, b_ref, nd_ref, out_ref):
    w = w_ref[...]
    half = a_ref.shape[2]
    acc = (jnp.dot(a_ref[0], w[:half], preferred_element_type=jnp.float32)
           + jnp.dot(a_ref[1], w[half:], preferred_element_type=jnp.float32))
    out_ref[...] = acc * nd_ref[...] + b_ref[...]


def _run_l2(agg, w2, b2, nd, npad, hid):
    return pl.pallas_call(
        _l2_body,
        grid=(npad // RB,),
        in_specs=[
            pl.BlockSpec((2, RB, hid // 2), lambda i: (0, i, 0)),
            pl.BlockSpec((hid, hid), lambda i: (0, 0)),
            pl.BlockSpec((1, hid), lambda i: (0, 0)),
            pl.BlockSpec((RB, 1), lambda i: (i, 0)),
        ],
        out_specs=pl.BlockSpec((RB, hid), lambda i: (i, 0)),
        out_shape=jax.ShapeDtypeStruct((npad, hid), jnp.float32),
    )(agg, w2, b2, nd)


def kernel(x, edge_index, W1, b1, W2, b2):
    n, fin = x.shape
    hid = W1.shape[1]
    e = edge_index.shape[1]

    npad = -(-n // RB) * RB
    if npad == n:
        npad += RB
    grp = 16 * K * NS * NC
    epad = -(-e // grp) * grp
    ech1 = epad // (K * NS * NC)
    ech2 = epad // (K * NS)

    src = edge_index[0].astype(jnp.int32)
    dst = edge_index[1].astype(jnp.int32)
    fill = n + (jnp.arange(epad - e, dtype=jnp.int32) % (npad - n))
    src_p = jnp.concatenate([src, fill])
    dst_p = jnp.concatenate([dst, fill])
    dst2 = dst_p.reshape(epad // K, K)
    xp = jnp.pad(x, ((0, npad - n), (0, 0)))

    degp = _run_degree(src_p, dst_p, npad, epad)
    xs, ns, nd = _run_prep(degp, xp, n, npad, fin)
    agg1 = _run_edge_pass(xs, src_p, dst2, npad, fin, ech1, False)
    hs = _run_l1(agg1, W1, b1.reshape(1, hid), nd, ns, npad, fin, hid)
    agg2 = _run_edge_pass(hs.reshape(NC * npad, hid // 2), src_p, dst2,
                          npad, hid // 2, ech2, True)
    out = _run_l2(agg2, W2, b2.reshape(1, hid), nd, npad, hid)
    return out[:n]


# speedup vs baseline: 10.4509x; 1.0042x over previous
import functools

import jax
import jax.numpy as jnp
from jax import lax
from jax.experimental import pallas as pl
from jax.experimental.pallas import tpu as pltpu
from jax.experimental.pallas import tpu_sc as plsc

NC = 2
NS = 16
L = 16
K = 128
RB = 2048


@functools.cache
def _degree_kernel(npad: int, epad: int):
    ed = epad // (NC * NS)

    @functools.partial(
        pl.kernel,
        out_type=jax.ShapeDtypeStruct((2, NC * NS, npad), jnp.float32),
        mesh=plsc.VectorSubcoreMesh(core_axis_name="c", subcore_axis_name="s"),
        compiler_params=pltpu.CompilerParams(needs_layout_passes=False),
        scratch_types=[
            pltpu.VMEM((ed,), jnp.int32),
            pltpu.VMEM((ed,), jnp.int32),
            pltpu.VMEM((npad,), jnp.float32),
            pltpu.VMEM((npad,), jnp.float32),
        ],
    )
    def deg(src_hbm, dst_hbm, out_hbm, src_v, dst_v, hsrc, hdst):
        c = lax.axis_index("c")
        s = lax.axis_index("s")
        w = s * NC + c
        pltpu.sync_copy(src_hbm.at[pl.ds(w * ed, ed)], src_v)
        pltpu.sync_copy(dst_hbm.at[pl.ds(w * ed, ed)], dst_v)

        zero = jnp.zeros((L,), jnp.float32)

        def zbody(i, _):
            hsrc[pl.ds(i * L, L)] = zero
            hdst[pl.ds(i * L, L)] = zero
            return 0

        lax.fori_loop(0, npad // L, zbody, 0)

        ones = jnp.full((L,), 1.0, jnp.float32)

        def ebody(i, _):
            plsc.addupdate_scatter(hsrc, [src_v[pl.ds(i * L, L)]], ones)
            plsc.addupdate_scatter(hdst, [dst_v[pl.ds(i * L, L)]], ones)
            return 0

        lax.fori_loop(0, ed // L, ebody, 0)

        pltpu.sync_copy(hsrc, out_hbm.at[0, w])
        pltpu.sync_copy(hdst, out_hbm.at[1, w])

    return deg


@functools.cache
def _edge_pass_kernel(npad: int, f: int, ech: int, feature_split: bool):
    rpw = npad // NS
    sch = 8
    nsc = ech // sch
    assert ech % sch == 0 and nsc % 2 == 0 and nsc >= 4

    @functools.partial(
        pl.kernel,
        out_type=jax.ShapeDtypeStruct((NC, npad, f), jnp.float32),
        mesh=plsc.VectorSubcoreMesh(core_axis_name="c", subcore_axis_name="s"),
        compiler_params=pltpu.CompilerParams(needs_layout_passes=False),
        scratch_types=[
            pltpu.VMEM((sch * K,), jnp.int32),
            pltpu.VMEM((sch, K), jnp.int32),
            pltpu.VMEM((sch * K,), jnp.int32),
            pltpu.VMEM((sch, K), jnp.int32),
            pltpu.VMEM((K, f), jnp.float32),
            pltpu.VMEM((K, f), jnp.float32),
            pltpu.VMEM_SHARED((npad, f), jnp.float32),
            pltpu.SemaphoreType.DMA,
            pltpu.SemaphoreType.DMA,
            pltpu.SemaphoreType.DMA,
        ],
    )
    def ep(tbl_hbm, srcb_hbm, dst2_hbm, out_hbm,
           src_a, dst_a, src_b, dst_b, buf0, buf1, acc, gsem, ssem, isem):
        c = lax.axis_index("c")
        s = lax.axis_index("s")
        if feature_split:
            tbl = tbl_hbm.at[pl.ds(c * npad, npad)]
            cbase = s * ech
        else:
            tbl = tbl_hbm
            cbase = (s * NC + c) * ech

        idx = [(src_a, dst_a), (src_b, dst_b)]
        bufs = [buf0, buf1]

        zero = jnp.zeros((L,), jnp.float32)

        def zbody(i, _):
            for k in range(f // L):
                buf0[i, pl.ds(k * L, L)] = zero
            return 0

        lax.fori_loop(0, K, zbody, 0)
        for t in range(rpw // K):
            pltpu.sync_copy(buf0, acc.at[pl.ds(s * rpw + t * K, K)])
        plsc.subcore_barrier()

        def i_start(m, p):
            sv, dv = idx[p]
            base = cbase + m * sch
            pltpu.async_copy(srcb_hbm.at[pl.ds(base * K, sch * K)], sv, isem)
            pltpu.async_copy(dst2_hbm.at[pl.ds(base, sch)], dv, isem)

        def i_wait(m, p):
            sv, dv = idx[p]
            base = cbase + m * sch
            pltpu.make_async_copy(
                srcb_hbm.at[pl.ds(base * K, sch * K)], sv, isem).wait()
            pltpu.make_async_copy(
                dst2_hbm.at[pl.ds(base, sch)], dv, isem).wait()

        def g_start(jj, p, bp):
            sv, _ = idx[p]
            pltpu.async_copy(
                tbl.at[sv.at[pl.ds(jj * K, K)]], bufs[bp], gsem)

        def g_wait(jj, p, bp):
            sv, _ = idx[p]
            pltpu.make_async_copy(
                tbl.at[sv.at[pl.ds(jj * K, K)]], bufs[bp], gsem).wait()

        def s_start(jj, p, bp):
            _, dv = idx[p]
            pltpu.async_copy(bufs[bp], acc.at[dv.at[jj]], ssem, add=True)

        def s_wait(jj, p, bp):
            _, dv = idx[p]
            pltpu.make_async_copy(bufs[bp], acc.at[dv.at[jj]], ssem).wait()

        def steady(jj, p, np_, pw):
            bp = jj % 2
            g_wait(jj, p, bp)
            s_wait(pw[0], pw[1], 1 - bp)
            g_start(np_[0], np_[1], 1 - bp)
            s_start(jj, p, bp)

        i_start(0, 0)
        i_wait(0, 0)
        i_start(1, 1)
        g_start(0, 0, 0)
        g_start(1, 0, 1)
        g_wait(0, 0, 0)
        s_start(0, 0, 0)
        for jj in range(1, sch - 1):
            steady(jj, 0, (jj + 1, 0), (jj - 1, 0))
        i_wait(1, 1)
        steady(sch - 1, 0, (0, 1), (sch - 2, 0))

        def spair(q, _):
            for (m, p) in ((2 * q + 1, 1), (2 * q + 2, 0)):
                bp0 = 0
                g_wait(0, p, bp0)
                s_wait(sch - 1, 1 - p, 1 - bp0)
                g_start(1, p, 1 - bp0)
                s_start(0, p, bp0)
                i_start(m + 1, 1 - p)
                for jj in range(1, sch - 1):
                    steady(jj, p, (jj + 1, p), (jj - 1, p))
                i_wait(m + 1, 1 - p)
                steady(sch - 1, p, (0, 1 - p), (sch - 2, p))
            return 0

        lax.fori_loop(0, (nsc - 2) // 2, spair, 0)

        p = 1
        g_wait(0, p, 0)
        s_wait(sch - 1, 0, 1)
        g_start(1, p, 1)
        s_start(0, p, 0)
        for jj in range(1, sch - 1):
            steady(jj, p, (jj + 1, p), (jj - 1, p))
        jl = sch - 1
        g_wait(jl, p, jl % 2)
        s_start(jl, p, jl % 2)
        s_wait(jl - 1, p, 1 - jl % 2)
        s_wait(jl, p, jl % 2)

        plsc.subcore_barrier()
        pltpu.sync_copy(acc.at[pl.ds(s * rpw, rpw)],
                        out_hbm.at[c, pl.ds(s * rpw, rpw)])

    return ep


def _run_degree(src_p, dst_p, npad, epad):
    return _degree_kernel(npad, epad)(src_p, dst_p)


def _run_edge_pass(tbl_flat, srcb, dst2, npad, f, ech, feature_split):
    return _edge_pass_kernel(npad, f, ech, feature_split)(tbl_flat, srcb, dst2)


def _prep_body(n, degp_ref, x_ref, xs_ref, ns_ref, nd_ref):
    d = jnp.sum(degp_ref[...], axis=1)[:, :n]
    ns = jnp.where(d[0] > 0, lax.rsqrt(d[0]), 0.0)
    nd = jnp.where(d[1] > 0, lax.rsqrt(d[1]), 0.0)
    ns_ref[...] = ns[:, None]
    nd_ref[...] = nd[:, None]
    xs_ref[...] = x_ref[...] * ns[:, None]


def _run_prep(degp, x, n, npad, fin):
    nw = degp.shape[1]
    npd = degp.shape[2]
    return pl.pallas_call(
        functools.partial(_prep_body, n),
        grid=(1,),
        in_specs=[
            pl.BlockSpec((2, nw, npd), lambda i: (0, 0, 0)),
            pl.BlockSpec((n, fin), lambda i: (0, 0)),
        ],
        out_specs=[
            pl.BlockSpec((n, fin), lambda i: (0, 0)),
            pl.BlockSpec((n, 1), lambda i: (0, 0)),
            pl.BlockSpec((n, 1), lambda i: (0, 0)),
        ],
        out_shape=[
            jax.ShapeDtypeStruct((npad, fin), jnp.float32),
            jax.ShapeDtypeStruct((npad, 1), jnp.float32),
            jax.ShapeDtypeStruct((npad, 1), jnp.float32),
        ],
    )(degp, x)


def _l1_body(a_ref, w_ref, b_ref, nd_ref, ns_ref, hs_ref):
    a = a_ref[0] + a_ref[1]
    acc = jnp.dot(a, w_ref[...], preferred_element_type=jnp.float32)
    h = acc * nd_ref[...] + b_ref[...]
    h = jnp.maximum(h, 0.0) * ns_ref[...]
    hh = h.shape[1] // 2
    hs_ref[0] = h[:, :hh]
    hs_ref[1] = h[:, hh:]


def _run_l1(agg, w1, b1, nd, ns, npad, fin, hid):
    return pl.pallas_call(
        _l1_body,
        grid=(npad // RB,),
        in_specs=[
            pl.BlockSpec((2, RB, fin), lambda i: (0, i, 0)),
            pl.BlockSpec((fin, hid), lambda i: (0, 0)),
            pl.BlockSpec((1, hid), lambda i: (0, 0)),
            pl.BlockSpec((RB, 1), lambda i: (i, 0)),
            pl.BlockSpec((RB, 1), lambda i: (i, 0)),
        ],
        out_specs=pl.BlockSpec((2, RB, hid // 2), lambda i: (0, i, 0)),
        out_shape=jax.ShapeDtypeStruct((2, npad, hid // 2), jnp.float32),
    )(agg, w1, b1, nd, ns)


def _l2_body(a_ref, w_ref, b_ref, nd_ref, out_ref):
    w = w_ref[...]
    half = a_ref.shape[2]
    acc = (jnp.dot(a_ref[0], w[:half], preferred_element_type=jnp.float32)
           + jnp.dot(a_ref[1], w[half:], preferred_element_type=jnp.float32))
    out_ref[...] = acc * nd_ref[...] + b_ref[...]


def _run_l2(agg, w2, b2, nd, npad, hid):
    return pl.pallas_call(
        _l2_body,
        grid=(npad // RB,),
        in_specs=[
            pl.BlockSpec((2, RB, hid // 2), lambda i: (0, i, 0)),
            pl.BlockSpec((hid, hid), lambda i: (0, 0)),
            pl.BlockSpec((1, hid), lambda i: (0, 0)),
            pl.BlockSpec((RB, 1), lambda i: (i, 0)),
        ],
        out_specs=pl.BlockSpec((RB, hid), lambda i: (i, 0)),
        out_shape=jax.ShapeDtypeStruct((npad, hid), jnp.float32),
    )(agg, w2, b2, nd)


def kernel(x, edge_index, W1, b1, W2, b2):
    n, fin = x.shape
    hid = W1.shape[1]
    e = edge_index.shape[1]

    npad = -(-n // RB) * RB
    if npad == n:
        npad += RB
    grp = 16 * K * NS * NC
    epad = -(-e // grp) * grp
    ech1 = epad // (K * NS * NC)
    ech2 = epad // (K * NS)

    src = edge_index[0].astype(jnp.int32)
    dst = edge_index[1].astype(jnp.int32)
    fill = n + (jnp.arange(epad - e, dtype=jnp.int32) % (npad - n))
    src_p = jnp.concatenate([src, fill])
    dst_p = jnp.concatenate([dst, fill])
    dst2 = dst_p.reshape(epad // K, K)

    degp = _run_degree(src_p, dst_p, npad, epad)
    xs, ns, nd = _run_prep(degp, x, n, npad, fin)
    agg1 = _run_edge_pass(xs, src_p, dst2, npad, fin, ech1, False)
    hs = _run_l1(agg1, W1, b1.reshape(1, hid), nd, ns, npad, fin, hid)
    agg2 = _run_edge_pass(hs.reshape(NC * npad, hid // 2), src_p, dst2,
                          npad, hid // 2, ech2, True)
    out = _run_l2(agg2, W2, b2.reshape(1, hid), nd, npad, hid)
    return out[:n]
